# Initial kernel scaffold; baseline (speedup 1.0000x reference)
import jax
import jax.numpy as jnp
from jax import lax
from jax.experimental import pallas as pl
from jax.experimental.pallas import tpu as pltpu
from jax.experimental.pallas import tpu_sc as plsc

N = 50000
E = 800000
H = 64
HH = 32
NG = 16
CFG = 24
NOPS = 120

CH = 128
SUP = 25
NSUP = 16
TILES = 16
EPT = CH * SUP * NSUP
E_PAD = EPT * TILES
ROWS_PAD = E_PAD // CH
ACC_ROWS = 51200
DUMP = N
ZROWS = 800
SLOTS = 3

BT = 1000
NB_TC = N // BT

_INTERPRET = False


def _agg_body(y_hbm, g_hbm, s_hbm, z_hbm, out_hbm, acc, gbuf, sbuf, rows, zbuf,
              gsem):
    c = lax.axis_index("c")
    s = lax.axis_index("s")
    pltpu.sync_copy(z_hbm, zbuf)
    for i in range(ACC_ROWS // TILES // ZROWS):
        pltpu.sync_copy(
            zbuf, acc.at[pl.ds(s * (ACC_ROWS // TILES) + i * ZROWS, ZROWS)])
    plsc.subcore_barrier()

    row_t = s * (SUP * NSUP)

    def sup_body(sup, carry):
        r0 = row_t + sup * SUP
        pltpu.sync_copy(g_hbm.at[pl.ds(c * ROWS_PAD + r0, SUP)], gbuf)
        pltpu.sync_copy(s_hbm.at[pl.ds(r0, SUP)], sbuf)
        handles = [None] * SLOTS

        def fire(k):
            sl = k % SLOTS
            handles[sl] = pltpu.async_copy(
                y_hbm.at[gbuf.at[k]], rows.at[sl], gsem.at[sl])

        def drain(w):
            sl = w % SLOTS
            handles[sl].wait()
            pltpu.sync_copy(rows.at[sl], acc.at[sbuf.at[w]], add=True)

        for k in range(SUP):
            fire(k)
            if k >= SLOTS - 1:
                drain(k - (SLOTS - 1))
        for w in range(SUP - SLOTS + 1, SUP):
            drain(w)
        return carry

    lax.fori_loop(0, NSUP, sup_body, 0)
    plsc.subcore_barrier()
    rpt = N // TILES
    pltpu.sync_copy(acc.at[pl.ds(s * rpt, rpt)],
                    out_hbm.at[pl.ds(c * N + s * rpt, rpt)])


def _cnt_body(c_hbm, z_hbm, o_hbm, out_hbm, acc, cbuf, ones, zbuf, csem):
    c = lax.axis_index("c")
    s = lax.axis_index("s")
    pltpu.sync_copy(z_hbm, zbuf)
    for i in range(ACC_ROWS // TILES // ZROWS):
        pltpu.sync_copy(
            zbuf, acc.at[pl.ds(s * (ACC_ROWS // TILES) + i * ZROWS, ZROWS)])
    pltpu.sync_copy(o_hbm, ones)
    plsc.subcore_barrier()

    row_t = s * (SUP * NSUP)

    def sup_body(sup, carry):
        r0 = row_t + sup * SUP
        pltpu.sync_copy(c_hbm.at[pl.ds(c * ROWS_PAD + r0, SUP)], cbuf)
        handles = []
        for k in range(SUP):
            handles.append(
                pltpu.async_copy(ones, acc.at[cbuf.at[k]], csem, add=True))
        for h in handles:
            h.wait()
        return carry

    lax.fori_loop(0, NSUP, sup_body, 0)
    plsc.subcore_barrier()
    rpt = N // TILES
    pltpu.sync_copy(acc.at[pl.ds(s * rpt, rpt)],
                    out_hbm.at[pl.ds(c * N + s * rpt, rpt)])


def _sc_mesh():
    return plsc.VectorSubcoreMesh(core_axis_name="c", subcore_axis_name="s")


def _agg(y_flat, garr, sarr, zeros32):
    return pl.kernel(
        _agg_body,
        out_type=jax.ShapeDtypeStruct((2 * N, HH), jnp.float32),
        mesh=_sc_mesh(),
        scratch_types=[
            pltpu.VMEM_SHARED((ACC_ROWS, HH), jnp.float32),
            pltpu.VMEM((SUP, CH), jnp.int32),
            pltpu.VMEM((SUP, CH), jnp.int32),
            pltpu.VMEM((SLOTS, CH, HH), jnp.float32),
            pltpu.VMEM((ZROWS, HH), jnp.float32),
            pltpu.SemaphoreType.DMA((SLOTS,)),
        ],
        interpret=_INTERPRET,
    )(y_flat, garr, sarr, zeros32)


def _cnt(carr, zeros16, ones16):
    return pl.kernel(
        _cnt_body,
        out_type=jax.ShapeDtypeStruct((2 * N, 16), jnp.float32),
        mesh=_sc_mesh(),
        scratch_types=[
            pltpu.VMEM_SHARED((ACC_ROWS, 16), jnp.float32),
            pltpu.VMEM((SUP, CH), jnp.int32),
            pltpu.VMEM((CH, 16), jnp.float32),
            pltpu.VMEM((ZROWS, 16), jnp.float32),
            pltpu.SemaphoreType.DMA,
        ],
        interpret=_INTERPRET,
    )(carr, zeros16, ones16)


def _dot(a, b):
    return jnp.dot(a, b, preferred_element_type=jnp.float32)


def _pre_body(nf_ref, op_ref, sh_ref, wf_ref, wb_ref, wr_ref, tof_ref, tob_ref,
              tor_ref, tsf_ref, tsb_ref, tsr_ref, br_ref,
              yf_ref, yb_ref, xr_ref):
    nf = nf_ref[...]
    opv = op_ref[0, 0, :]
    shv = sh_ref[0, 0, :]
    ohop = (opv[:, None] == lax.broadcasted_iota(jnp.int32, (BT, 128), 1)
            ).astype(jnp.float32)
    ohsh = (shv[:, None] == lax.broadcasted_iota(jnp.int32, (BT, 8), 1)
            ).astype(jnp.float32)

    def proj(w_ref, to_ref, ts_ref):
        return (_dot(nf, w_ref[...]) + _dot(ohop, to_ref[...])
                + _dot(ohsh, ts_ref[...]))

    yf = proj(wf_ref, tof_ref, tsf_ref)
    yb = proj(wb_ref, tob_ref, tsb_ref)
    xr = proj(wr_ref, tor_ref, tsr_ref) + br_ref[...]
    yf_ref[0:1] = yf[:, :HH][None]
    yf_ref[1:2] = yf[:, HH:][None]
    yb_ref[0:1] = yb[:, :HH][None]
    yb_ref[1:2] = yb[:, HH:][None]
    xr_ref[...] = xr


def _combine(af_ref, ab_ref, xr_ref, cin_ref, cout_ref):
    af = af_ref[...]
    ab = ab_ref[...]
    rin = 1.0 / jnp.maximum(cin_ref[...][:, 0:1], 1.0)
    rout = 1.0 / jnp.maximum(cout_ref[...][:, 0:1], 1.0)
    xlo = af[0] * rin + ab[0] * rout
    xhi = af[1] * rin + ab[1] * rout
    x = jnp.concatenate([xlo, xhi], axis=1) + xr_ref[...]
    return jnp.maximum(x, 0.0)


def _mid_body(af_ref, ab_ref, xr_ref, cin_ref, cout_ref, wf_ref, wb_ref,
              wr_ref, br_ref, yf_ref, yb_ref, xr_o_ref):
    x = _combine(af_ref, ab_ref, xr_ref, cin_ref, cout_ref)
    yf = _dot(x, wf_ref[...])
    yb = _dot(x, wb_ref[...])
    xr = _dot(x, wr_ref[...]) + br_ref[...]
    yf_ref[0:1] = yf[:, :HH][None]
    yf_ref[1:2] = yf[:, HH:][None]
    yb_ref[0:1] = yb[:, :HH][None]
    yb_ref[1:2] = yb[:, HH:][None]
    xr_o_ref[...] = xr


def _last_body(af_ref, ab_ref, xr_ref, cin_ref, cout_ref, x_ref):
    x_ref[...] = _combine(af_ref, ab_ref, xr_ref, cin_ref, cout_ref)


def _pool_body(x_ref, b3_ref, cfg_ref, w1a_ref, w1b_ref, b1_ref, w2_ref,
               b2_ref, out_ref, sums, maxs, cnts):
    i = pl.program_id(0)

    @pl.when(i == 0)
    def _init():
        sums[...] = jnp.zeros((NG, H), jnp.float32)
        maxs[...] = jnp.full((NG, H), -3e38, jnp.float32)
        cnts[...] = jnp.zeros((NG, H), jnp.float32)

    x = x_ref[...]
    bv = b3_ref[0, 0, :]
    oh = (bv[:, None] == lax.broadcasted_iota(jnp.int32, (BT, NG), 1)
          ).astype(jnp.float32)
    sums[...] += lax.dot_general(oh, x, (((0,), (0,)), ((), ())),
                                 preferred_element_type=jnp.float32)
    cnts[...] += jnp.broadcast_to(jnp.sum(oh, axis=0)[:, None], (NG, H))
    mx = maxs[...]
    upd = []
    for g in range(NG):
        mg = jnp.max(jnp.where((bv == g)[:, None], x, -3e38), axis=0)
        upd.append(jnp.maximum(mx[g], mg))
    maxs[...] = jnp.stack(upd, axis=0)

    @pl.when(i == NB_TC - 1)
    def _final():
        avg = sums[...] / jnp.maximum(cnts[...], 1.0)
        g128 = jnp.concatenate([avg, maxs[...]], axis=1)
        h = jnp.maximum(
            _dot(g128, w1a_ref[...]) + _dot(cfg_ref[...], w1b_ref[...])
            + b1_ref[...], 0.0)
        out_ref[...] = _dot(h, w2_ref[...]) + b2_ref[...]


def _full(shape):
    return pl.BlockSpec(shape, lambda i: (0,) * len(shape))


_NODE140 = pl.BlockSpec((BT, 140), lambda i: (i, 0))
_IDX3 = pl.BlockSpec((1, 1, BT), lambda i: (i, 0, 0))
_HALF3 = pl.BlockSpec((2, BT, HH), lambda i: (0, i, 0))
_NODE64 = pl.BlockSpec((BT, H), lambda i: (i, 0))
_CNT16 = pl.BlockSpec((BT, 16), lambda i: (i, 0))


def _pre(nf, op3, sh3, wf, wb, wr, tof, tob, tor, tsf, tsb, tsr, br):
    half = jax.ShapeDtypeStruct((2, N, HH), jnp.float32)
    return pl.pallas_call(
        _pre_body,
        grid=(NB_TC,),
        in_specs=[_NODE140, _IDX3, _IDX3,
                  _full((140, H)), _full((140, H)), _full((140, H)),
                  _full((128, H)), _full((128, H)), _full((128, H)),
                  _full((8, H)), _full((8, H)), _full((8, H)),
                  _full((1, H))],
        out_specs=[_HALF3, _HALF3, _NODE64],
        out_shape=[half, half, jax.ShapeDtypeStruct((N, H), jnp.float32)],
        interpret=_INTERPRET,
    )(nf, op3, sh3, wf, wb, wr, tof, tob, tor, tsf, tsb, tsr, br)


def _mid(aggf, aggb, xr, cin, cout, wf, wb, wr, br):
    half = jax.ShapeDtypeStruct((2, N, HH), jnp.float32)
    return pl.pallas_call(
        _mid_body,
        grid=(NB_TC,),
        in_specs=[_HALF3, _HALF3, _NODE64, _CNT16, _CNT16,
                  _full((H, H)), _full((H, H)), _full((H, H)), _full((1, H))],
        out_specs=[_HALF3, _HALF3, _NODE64],
        out_shape=[half, half, jax.ShapeDtypeStruct((N, H), jnp.float32)],
        interpret=_INTERPRET,
    )(aggf, aggb, xr, cin, cout, wf, wb, wr, br)


def _last(aggf, aggb, xr, cin, cout):
    return pl.pallas_call(
        _last_body,
        grid=(NB_TC,),
        in_specs=[_HALF3, _HALF3, _NODE64, _CNT16, _CNT16],
        out_specs=[_NODE64],
        out_shape=[jax.ShapeDtypeStruct((N, H), jnp.float32)],
        interpret=_INTERPRET,
    )(aggf, aggb, xr, cin, cout)[0]


def _pool(x3, b3, cfg, w1a, w1b, b1, w2, b2):
    return pl.pallas_call(
        _pool_body,
        grid=(NB_TC,),
        in_specs=[_NODE64, _IDX3, _full((NG, CFG)), _full((2 * H, H)),
                  _full((CFG, H)), _full((1, H)), _full((H, 1)),
                  _full((1, 1))],
        out_specs=pl.BlockSpec((NG, 1), lambda i: (0, 0)),
        out_shape=jax.ShapeDtypeStruct((NG, 1), jnp.float32),
        scratch_shapes=[pltpu.VMEM((NG, H), jnp.float32)] * 3,
        interpret=_INTERPRET,
    )(x3, b3, cfg, w1a, w1b, b1, w2, b2)


def kernel(node_feat, node_opcode, edge_index, config_feat, n_configs, batch,
           params):
    f32 = jnp.float32
    shape_idx = node_feat[:, -1].astype(jnp.int32)
    op3 = node_opcode.reshape(NB_TC, 1, BT)
    sh3 = shape_idx.reshape(NB_TC, 1, BT)
    b3 = batch.reshape(NB_TC, 1, BT)

    L = params["layers"]
    op_emb = params["op_emb"]
    sh_emb = params["shape_emb"]

    def prep0(w):
        w140 = jnp.concatenate([w[:139], jnp.zeros((1, H), f32)], axis=0)
        top = jnp.zeros((128, H), f32).at[:NOPS].set(op_emb @ w[139:147])
        tsh = sh_emb @ w[147:151]
        return w140, top, tsh

    wf0, tof, tsf = prep0(L[0]["Wn_f"])
    wb0, tob, tsb = prep0(L[0]["Wn_b"])
    wr0, tor, tsr = prep0(L[0]["Wr_f"] + L[0]["Wr_b"])
    br0 = (L[0]["b_f"] + L[0]["b_b"])[None]

    src = edge_index[:, 0]
    dst = edge_index[:, 1]
    pad_g = jnp.zeros((E_PAD - E,), jnp.int32)
    pad_s = jnp.full((E_PAD - E,), DUMP, jnp.int32)
    srcg = jnp.concatenate([src, pad_g])
    dstg = jnp.concatenate([dst, pad_g])
    srcs = jnp.concatenate([src, pad_s])
    dsts = jnp.concatenate([dst, pad_s])
    garrF = jnp.concatenate([srcg, srcg + N]).reshape(2 * ROWS_PAD, CH)
    garrB = jnp.concatenate([dstg, dstg + N]).reshape(2 * ROWS_PAD, CH)
    sarrF = dsts.reshape(ROWS_PAD, CH)
    sarrB = srcs.reshape(ROWS_PAD, CH)
    carr = jnp.concatenate([dsts, srcs]).reshape(2 * ROWS_PAD, CH)
    zeros32 = jnp.zeros((ZROWS, HH), f32)
    zeros16 = jnp.zeros((ZROWS, 16), f32)
    ones16 = jnp.ones((CH, 16), f32)

    cnt = _cnt(carr, zeros16, ones16)
    cin = cnt[:N]
    cout = cnt[N:]

    yf, yb, xr = _pre(node_feat, op3, sh3, wf0, wb0, wr0, tof, tob, tor,
                      tsf, tsb, tsr, br0)
    x3 = None
    for l in range(3):
        aggf = _agg(yf.reshape(2 * N, HH), garrF, sarrF, zeros32)
        aggb = _agg(yb.reshape(2 * N, HH), garrB, sarrB, zeros32)
        aggf = aggf.reshape(2, N, HH)
        aggb = aggb.reshape(2, N, HH)
        if l < 2:
            lay = L[l + 1]
            yf, yb, xr = _mid(aggf, aggb, xr, cin, cout, lay["Wn_f"],
                              lay["Wn_b"], lay["Wr_f"] + lay["Wr_b"],
                              (lay["b_f"] + lay["b_b"])[None])
        else:
            x3 = _last(aggf, aggb, xr, cin, cout)

    w1 = params["W1"]
    out = _pool(x3, b3, config_feat, w1[:2 * H], w1[2 * H:],
                params["b1"][None], params["W2"], params["b2"][None])
    return out[:, 0]


# speedup vs baseline: 4.4239x; 4.4239x over previous
import jax
import jax.numpy as jnp
from jax import lax
from jax.experimental import pallas as pl
from jax.experimental.pallas import tpu as pltpu
from jax.experimental.pallas import tpu_sc as plsc

N = 50000
E = 800000
H = 64
HH = 32
NG = 16
CFG = 24
NOPS = 120

CH = 128
SUP = 24
NSUP = 17
TILES = 16
EPT = CH * SUP * NSUP
E_PAD = EPT * TILES
ROWS_PAD = E_PAD // CH
ACC_ROWS = 51200
DUMP = N
ZROWS = 80
SLOTS = 3

BT = 1000
NB_TC = N // BT

_INTERPRET = False


def _agg_body(y_hbm, g_hbm, s_hbm, z_hbm, out_hbm, acc, gbuf, sbuf, rows, zbuf,
              gsem):
    c = lax.axis_index("c")
    s = lax.axis_index("s")
    pltpu.sync_copy(z_hbm, zbuf)
    for i in range(ACC_ROWS // TILES // ZROWS):
        pltpu.sync_copy(
            zbuf, acc.at[pl.ds(s * (ACC_ROWS // TILES) + i * ZROWS, ZROWS)])
    plsc.subcore_barrier()

    row_t = s * (SUP * NSUP)

    def sup_body(sup, carry):
        r0 = row_t + sup * SUP
        pltpu.sync_copy(g_hbm.at[pl.ds(c * ROWS_PAD + r0, SUP)], gbuf)
        pltpu.sync_copy(s_hbm.at[pl.ds(r0, SUP)], sbuf)
        handles = [None] * SLOTS

        def fire(k):
            sl = k % SLOTS
            handles[sl] = pltpu.async_copy(
                y_hbm.at[gbuf.at[k]], rows.at[sl], gsem.at[sl])

        def drain(w):
            sl = w % SLOTS
            handles[sl].wait()
            pltpu.sync_copy(rows.at[sl], acc.at[sbuf.at[w]], add=True)

        for k in range(SUP):
            fire(k)
            if k >= SLOTS - 1:
                drain(k - (SLOTS - 1))
        for w in range(SUP - SLOTS + 1, SUP):
            drain(w)
        return carry

    lax.fori_loop(0, NSUP, sup_body, 0)
    plsc.subcore_barrier()
    rpt = ACC_ROWS // TILES
    pltpu.sync_copy(acc.at[pl.ds(s * rpt, rpt)],
                    out_hbm.at[pl.ds(c * ACC_ROWS + s * rpt, rpt)])


def _cnt_body(c_hbm, z_hbm, o_hbm, out_hbm, acc, cbuf, ones, zbuf, csem):
    c = lax.axis_index("c")
    s = lax.axis_index("s")
    pltpu.sync_copy(z_hbm, zbuf)
    for i in range(ACC_ROWS // TILES // ZROWS):
        pltpu.sync_copy(
            zbuf, acc.at[pl.ds(s * (ACC_ROWS // TILES) + i * ZROWS, ZROWS)])
    pltpu.sync_copy(o_hbm, ones)
    plsc.subcore_barrier()

    row_t = s * (SUP * NSUP)

    def sup_body(sup, carry):
        r0 = row_t + sup * SUP
        pltpu.sync_copy(c_hbm.at[pl.ds(c * ROWS_PAD + r0, SUP)], cbuf)
        handles = []
        for k in range(SUP):
            handles.append(
                pltpu.async_copy(ones, acc.at[cbuf.at[k]], csem, add=True))
        for h in handles:
            h.wait()
        return carry

    lax.fori_loop(0, NSUP, sup_body, 0)
    plsc.subcore_barrier()
    rpt = ACC_ROWS // TILES
    pltpu.sync_copy(acc.at[pl.ds(s * rpt, rpt)],
                    out_hbm.at[pl.ds(c * ACC_ROWS + s * rpt, rpt)])


def _sc_mesh():
    return plsc.VectorSubcoreMesh(core_axis_name="c", subcore_axis_name="s")


def _agg(y_flat, garr, sarr, zeros32):
    return pl.kernel(
        _agg_body,
        out_type=jax.ShapeDtypeStruct((2 * ACC_ROWS, HH), jnp.float32),
        mesh=_sc_mesh(),
        scratch_types=[
            pltpu.VMEM_SHARED((ACC_ROWS, HH), jnp.float32),
            pltpu.VMEM((SUP, CH), jnp.int32),
            pltpu.VMEM((SUP, CH), jnp.int32),
            pltpu.VMEM((SLOTS, CH, HH), jnp.float32),
            pltpu.VMEM((ZROWS, HH), jnp.float32),
            pltpu.SemaphoreType.DMA((SLOTS,)),
        ],
        compiler_params=pltpu.CompilerParams(use_tc_tiling_on_sc=False),
        interpret=_INTERPRET,
    )(y_flat, garr, sarr, zeros32)


def _cnt(carr, zeros16, ones16):
    return pl.kernel(
        _cnt_body,
        out_type=jax.ShapeDtypeStruct((2 * ACC_ROWS, 16), jnp.float32),
        mesh=_sc_mesh(),
        scratch_types=[
            pltpu.VMEM_SHARED((ACC_ROWS, 16), jnp.float32),
            pltpu.VMEM((SUP, CH), jnp.int32),
            pltpu.VMEM((CH, 16), jnp.float32),
            pltpu.VMEM((ZROWS, 16), jnp.float32),
            pltpu.SemaphoreType.DMA,
        ],
        compiler_params=pltpu.CompilerParams(use_tc_tiling_on_sc=False),
        interpret=_INTERPRET,
    )(carr, zeros16, ones16)


def _dot(a, b):
    return jnp.dot(a, b, preferred_element_type=jnp.float32)


def _pre_body(nf_ref, op_ref, sh_ref, wf_ref, wb_ref, wr_ref, tof_ref, tob_ref,
              tor_ref, tsf_ref, tsb_ref, tsr_ref, br_ref,
              yf_ref, yb_ref, xr_ref):
    nf = nf_ref[...]
    opv = op_ref[0, 0, :]
    shv = sh_ref[0, 0, :]
    ohop = (opv[:, None] == lax.broadcasted_iota(jnp.int32, (BT, 128), 1)
            ).astype(jnp.float32)
    ohsh = (shv[:, None] == lax.broadcasted_iota(jnp.int32, (BT, 8), 1)
            ).astype(jnp.float32)

    def proj(w_ref, to_ref, ts_ref):
        return (_dot(nf, w_ref[...]) + _dot(ohop, to_ref[...])
                + _dot(ohsh, ts_ref[...]))

    yf = proj(wf_ref, tof_ref, tsf_ref)
    yb = proj(wb_ref, tob_ref, tsb_ref)
    xr = proj(wr_ref, tor_ref, tsr_ref) + br_ref[...]
    yf_ref[0:1] = yf[:, :HH][None]
    yf_ref[1:2] = yf[:, HH:][None]
    yb_ref[0:1] = yb[:, :HH][None]
    yb_ref[1:2] = yb[:, HH:][None]
    xr_ref[...] = xr


def _combine(af_ref, ab_ref, xr_ref, cin_ref, cout_ref):
    af = af_ref[...]
    ab = ab_ref[...]
    rin = 1.0 / jnp.maximum(cin_ref[0, :, 0:1], 1.0)
    rout = 1.0 / jnp.maximum(cout_ref[0, :, 0:1], 1.0)
    xlo = af[0] * rin + ab[0] * rout
    xhi = af[1] * rin + ab[1] * rout
    x = jnp.concatenate([xlo, xhi], axis=1) + xr_ref[...]
    return jnp.maximum(x, 0.0)


def _mid_body(af_ref, ab_ref, xr_ref, cin_ref, cout_ref, wf_ref, wb_ref,
              wr_ref, br_ref, yf_ref, yb_ref, xr_o_ref):
    x = _combine(af_ref, ab_ref, xr_ref, cin_ref, cout_ref)
    yf = _dot(x, wf_ref[...])
    yb = _dot(x, wb_ref[...])
    xr = _dot(x, wr_ref[...]) + br_ref[...]
    yf_ref[0:1] = yf[:, :HH][None]
    yf_ref[1:2] = yf[:, HH:][None]
    yb_ref[0:1] = yb[:, :HH][None]
    yb_ref[1:2] = yb[:, HH:][None]
    xr_o_ref[...] = xr


def _last_body(af_ref, ab_ref, xr_ref, cin_ref, cout_ref, x_ref):
    x_ref[...] = _combine(af_ref, ab_ref, xr_ref, cin_ref, cout_ref)


def _pool_body(x_ref, b3_ref, cfg_ref, w1a_ref, w1b_ref, b1_ref, w2_ref,
               b2_ref, out_ref, sums, maxs, cnts):
    i = pl.program_id(0)

    @pl.when(i == 0)
    def _init():
        sums[...] = jnp.zeros((NG, H), jnp.float32)
        maxs[...] = jnp.full((NG, H), -3e38, jnp.float32)
        cnts[...] = jnp.zeros((NG, H), jnp.float32)

    x = x_ref[...]
    bv = b3_ref[0, 0, :]
    oh = (bv[:, None] == lax.broadcasted_iota(jnp.int32, (BT, NG), 1)
          ).astype(jnp.float32)
    sums[...] += lax.dot_general(oh, x, (((0,), (0,)), ((), ())),
                                 preferred_element_type=jnp.float32)
    cnts[...] += jnp.broadcast_to(jnp.sum(oh, axis=0)[:, None], (NG, H))
    mx = maxs[...]
    upd = []
    for g in range(NG):
        mg = jnp.max(jnp.where(oh[:, g:g + 1] > 0.5, x, -3e38), axis=0)
        upd.append(jnp.maximum(mx[g], mg))
    maxs[...] = jnp.stack(upd, axis=0)

    @pl.when(i == NB_TC - 1)
    def _final():
        avg = sums[...] / jnp.maximum(cnts[...], 1.0)
        g128 = jnp.concatenate([avg, maxs[...]], axis=1)
        h = jnp.maximum(
            _dot(g128, w1a_ref[...]) + _dot(cfg_ref[...], w1b_ref[...])
            + b1_ref[...], 0.0)
        out_ref[...] = _dot(h, w2_ref[...]) + b2_ref[...]


def _full(shape):
    return pl.BlockSpec(shape, lambda i: (0,) * len(shape))


_NODE140 = pl.BlockSpec((BT, 140), lambda i: (i, 0))
_IDX3 = pl.BlockSpec((1, 1, BT), lambda i: (i, 0, 0))
_HALF3 = pl.BlockSpec((2, BT, HH), lambda i: (0, i, 0))
_NODE64 = pl.BlockSpec((BT, H), lambda i: (i, 0))
_CIN = pl.BlockSpec((1, BT, 16), lambda i: (0, i, 0))
_COUT = pl.BlockSpec((1, BT, 16), lambda i: (1, i, 0))


def _pre(nf, op3, sh3, wf, wb, wr, tof, tob, tor, tsf, tsb, tsr, br):
    half = jax.ShapeDtypeStruct((2, N, HH), jnp.float32)
    return pl.pallas_call(
        _pre_body,
        grid=(NB_TC,),
        in_specs=[_NODE140, _IDX3, _IDX3,
                  _full((140, H)), _full((140, H)), _full((140, H)),
                  _full((128, H)), _full((128, H)), _full((128, H)),
                  _full((8, H)), _full((8, H)), _full((8, H)),
                  _full((1, H))],
        out_specs=[_HALF3, _HALF3, _NODE64],
        out_shape=[half, half, jax.ShapeDtypeStruct((N, H), jnp.float32)],
        interpret=_INTERPRET,
    )(nf, op3, sh3, wf, wb, wr, tof, tob, tor, tsf, tsb, tsr, br)


def _mid(aggf, aggb, xr, cin, cout, wf, wb, wr, br):
    half = jax.ShapeDtypeStruct((2, N, HH), jnp.float32)
    return pl.pallas_call(
        _mid_body,
        grid=(NB_TC,),
        in_specs=[_HALF3, _HALF3, _NODE64, _CIN, _COUT,
                  _full((H, H)), _full((H, H)), _full((H, H)), _full((1, H))],
        out_specs=[_HALF3, _HALF3, _NODE64],
        out_shape=[half, half, jax.ShapeDtypeStruct((N, H), jnp.float32)],
        interpret=_INTERPRET,
    )(aggf, aggb, xr, cin, cout, wf, wb, wr, br)


def _last(aggf, aggb, xr, cin, cout):
    return pl.pallas_call(
        _last_body,
        grid=(NB_TC,),
        in_specs=[_HALF3, _HALF3, _NODE64, _CIN, _COUT],
        out_specs=[_NODE64],
        out_shape=[jax.ShapeDtypeStruct((N, H), jnp.float32)],
        interpret=_INTERPRET,
    )(aggf, aggb, xr, cin, cout)[0]


def _pool(x3, b3, cfg, w1a, w1b, b1, w2, b2):
    return pl.pallas_call(
        _pool_body,
        grid=(NB_TC,),
        in_specs=[_NODE64, _IDX3, _full((NG, CFG)), _full((2 * H, H)),
                  _full((CFG, H)), _full((1, H)), _full((H, 1)),
                  _full((1, 1))],
        out_specs=pl.BlockSpec((NG, 1), lambda i: (0, 0)),
        out_shape=jax.ShapeDtypeStruct((NG, 1), jnp.float32),
        scratch_shapes=[pltpu.VMEM((NG, H), jnp.float32)] * 3,
        interpret=_INTERPRET,
    )(x3, b3, cfg, w1a, w1b, b1, w2, b2)


def kernel(node_feat, node_opcode, edge_index, config_feat, n_configs, batch,
           params):
    f32 = jnp.float32
    shape_idx = node_feat[:, -1].astype(jnp.int32)
    op3 = node_opcode.reshape(NB_TC, 1, BT)
    sh3 = shape_idx.reshape(NB_TC, 1, BT)
    b3 = batch.reshape(NB_TC, 1, BT)

    L = params["layers"]
    op_emb = params["op_emb"]
    sh_emb = params["shape_emb"]

    def prep0(w):
        w140 = jnp.concatenate([w[:139], jnp.zeros((1, H), f32)], axis=0)
        top = jnp.zeros((128, H), f32).at[:NOPS].set(op_emb @ w[139:147])
        tsh = sh_emb @ w[147:151]
        return w140, top, tsh

    wf0, tof, tsf = prep0(L[0]["Wn_f"])
    wb0, tob, tsb = prep0(L[0]["Wn_b"])
    wr0, tor, tsr = prep0(L[0]["Wr_f"] + L[0]["Wr_b"])
    br0 = (L[0]["b_f"] + L[0]["b_b"])[None]

    src = edge_index[:, 0]
    dst = edge_index[:, 1]
    pad_g = jnp.zeros((E_PAD - E,), jnp.int32)
    pad_s = jnp.full((E_PAD - E,), DUMP, jnp.int32)
    srcg = jnp.concatenate([src, pad_g])
    dstg = jnp.concatenate([dst, pad_g])
    srcs = jnp.concatenate([src, pad_s])
    dsts = jnp.concatenate([dst, pad_s])
    garrF = jnp.concatenate([srcg, srcg + N]).reshape(2 * ROWS_PAD, CH)
    garrB = jnp.concatenate([dstg, dstg + N]).reshape(2 * ROWS_PAD, CH)
    sarrF = dsts.reshape(ROWS_PAD, CH)
    sarrB = srcs.reshape(ROWS_PAD, CH)
    carr = jnp.concatenate([dsts, srcs]).reshape(2 * ROWS_PAD, CH)
    zeros32 = jnp.zeros((ZROWS, HH), f32)
    zeros16 = jnp.zeros((ZROWS, 16), f32)
    ones16 = jnp.ones((CH, 16), f32)

    cnt = _cnt(carr, zeros16, ones16).reshape(2, ACC_ROWS, 16)

    yf, yb, xr = _pre(node_feat, op3, sh3, wf0, wb0, wr0, tof, tob, tor,
                      tsf, tsb, tsr, br0)
    x3 = None
    for l in range(3):
        aggf = _agg(yf.reshape(2 * N, HH), garrF, sarrF, zeros32)
        aggb = _agg(yb.reshape(2 * N, HH), garrB, sarrB, zeros32)
        aggf = aggf.reshape(2, ACC_ROWS, HH)
        aggb = aggb.reshape(2, ACC_ROWS, HH)
        if l < 2:
            lay = L[l + 1]
            yf, yb, xr = _mid(aggf, aggb, xr, cnt, cnt, lay["Wn_f"],
                              lay["Wn_b"], lay["Wr_f"] + lay["Wr_b"],
                              (lay["b_f"] + lay["b_b"])[None])
        else:
            x3 = _last(aggf, aggb, xr, cnt, cnt)

    w1 = params["W1"]
    out = _pool(x3, b3, config_feat, w1[:2 * H], w1[2 * H:],
                params["b1"][None], params["W2"], params["b2"][None])
    return out[:, 0]
